# baseline (speedup 1.0000x reference)
import functools

import jax
import jax.numpy as jnp
from jax import lax
from jax.experimental import pallas as pl
from jax.experimental.pallas import tpu as pltpu
from jax.experimental.pallas import tpu_sc as plsc

N = 10000
E = 320000
NP = 10240
BR = 512
NB = NP // BR
NG = 64
C = 128
NT = 16
K2 = -(-E // (2 * NT * C)) + (-(-E // (2 * NT * C)) % 2)
EP2 = 2 * NT * K2 * C
NPS = NP // NT
W = 8

_f32 = jnp.float32


@functools.lru_cache(maxsize=None)
def _make_agg_full():
    mesh = plsc.VectorSubcoreMesh(core_axis_name="c", subcore_axis_name="s")

    def body(g, src3, dst3, zz, out0, out1, idxs_v, idxd_v, bufa,
             s_sp, gsa):
        c = lax.axis_index("c")
        s = lax.axis_index("s")
        w = c * NT + s
        for r in range(NPS // C):
            pltpu.sync_copy(zz, s_sp.at[pl.ds(s * NPS + r * C, C)])
        pltpu.sync_copy(src3.at[w], idxs_v)
        pltpu.sync_copy(dst3.at[w], idxd_v)
        plsc.subcore_barrier()

        def step(j, carry):
            pltpu.async_copy(g.at[idxs_v.at[j]], bufa, gsa).wait()
            pltpu.sync_copy(bufa, s_sp.at[idxd_v.at[j]], add=True)
            return carry

        lax.fori_loop(0, K2, step, 0)
        plsc.subcore_barrier()

        @pl.when(c == 0)
        def _():
            pltpu.sync_copy(s_sp.at[pl.ds(s * NPS, NPS)],
                            out0.at[pl.ds(s * NPS, NPS)])

        @pl.when(c == 1)
        def _():
            pltpu.sync_copy(s_sp.at[pl.ds(s * NPS, NPS)],
                            out1.at[pl.ds(s * NPS, NPS)])

    return pl.kernel(
        body,
        out_type=[jax.ShapeDtypeStruct((NP, 128), _f32),
                  jax.ShapeDtypeStruct((NP, 128), _f32)],
        mesh=mesh,
        scratch_types=[
            pltpu.VMEM((K2, C), jnp.int32),
            pltpu.VMEM((K2, C), jnp.int32),
            pltpu.VMEM((C, 128), _f32),
            pltpu.VMEM_SHARED((NP, 128), _f32),
            pltpu.SemaphoreType.DMA,
        ],
    )


def _agg_full(g, src3, dst3, zz):
    return _make_agg_full()(g, src3, dst3, zz)


def _rspec(w):
    return pl.BlockSpec((BR, w), lambda i: (i, 0))


def _fullspec(r, w):
    return pl.BlockSpec((r, w), lambda i: (0, 0))


def _tc1_body(x_ref, w_ref, d0_ref, d1_ref, g_ref, dv_ref):
    deg = d0_ref[:, 0:1] + d1_ref[:, 0:1] + 1.0
    dinv = lax.rsqrt(jnp.maximum(deg, 1e-12))
    h = jnp.dot(x_ref[:], w_ref[:], preferred_element_type=_f32)
    g_ref[:] = dinv * h
    dv_ref[:] = jnp.broadcast_to(dinv, (BR, 8))


def _tc1(data_p, w1, d0, d1):
    return pl.pallas_call(
        _tc1_body,
        grid=(NB,),
        in_specs=[_rspec(128), _fullspec(128, 128), _rspec(128), _rspec(128)],
        out_specs=[_rspec(128), _rspec(8)],
        out_shape=[jax.ShapeDtypeStruct((NP, 128), _f32),
                   jax.ShapeDtypeStruct((NP, 8), _f32)],
    )(data_p, w1, d0, d1)


def _tc2_body(s0_ref, s1_ref, g_ref, dv_ref, b_ref, w_ref, ol_ref, oh_ref):
    dinv = dv_ref[:, 0:1]
    x = dinv * (s0_ref[:] + s1_ref[:] + g_ref[:]) + b_ref[:]
    x = jnp.maximum(x, 0.0)
    h = jnp.dot(x, w_ref[:], preferred_element_type=_f32)
    g2 = dinv * h
    ol_ref[:] = g2[:, :128]
    oh_ref[:] = g2[:, 128:]


def _tc2(s0, s1, g, dv, b, w):
    return pl.pallas_call(
        _tc2_body,
        grid=(NB,),
        in_specs=[_rspec(128), _rspec(128), _rspec(128), _rspec(8),
                  _fullspec(1, 128), _fullspec(128, 256)],
        out_specs=[_rspec(128), _rspec(128)],
        out_shape=[jax.ShapeDtypeStruct((NP, 128), _f32),
                   jax.ShapeDtypeStruct((NP, 128), _f32)],
    )(s0, s1, g, dv, b, w)


def _tc3_body(sla_ref, slb_ref, sha_ref, shb_ref, gl_ref, gh_ref, dv_ref,
              b_ref, w_ref, o_ref):
    dinv = dv_ref[:, 0:1]
    x_lo = jnp.maximum(
        dinv * (sla_ref[:] + slb_ref[:] + gl_ref[:]) + b_ref[:, :128], 0.0)
    x_hi = jnp.maximum(
        dinv * (sha_ref[:] + shb_ref[:] + gh_ref[:]) + b_ref[:, 128:], 0.0)
    h = (jnp.dot(x_lo, w_ref[:128, :], preferred_element_type=_f32)
         + jnp.dot(x_hi, w_ref[128:, :], preferred_element_type=_f32))
    o_ref[:] = dinv * h


def _tc3(sla, slb, sha, shb, gl, gh, dv, b, w):
    return pl.pallas_call(
        _tc3_body,
        grid=(NB,),
        in_specs=[_rspec(128), _rspec(128), _rspec(128), _rspec(128),
                  _rspec(128), _rspec(128), _rspec(8), _fullspec(1, 256),
                  _fullspec(256, 128)],
        out_specs=_rspec(128),
        out_shape=jax.ShapeDtypeStruct((NP, 128), _f32),
    )(sla, slb, sha, shb, gl, gh, dv, b, w)


def _tc4_body(s0_ref, s1_ref, g_ref, dv_ref, b_ref, batch_ref, out_ref):
    i = pl.program_id(0)

    @pl.when(i == 0)
    def _():
        out_ref[:] = jnp.full((NG, 128), -jnp.inf, _f32)

    dinv = dv_ref[:, 0:1]
    h = dinv * (s0_ref[:] + s1_ref[:] + g_ref[:]) + b_ref[:]
    batch = batch_ref[:]
    gid_iota = lax.broadcasted_iota(jnp.int32, (NG, 1), 0)

    def step(gid, acc):
        m = batch == gid
        red = jnp.max(jnp.where(m, h, -jnp.inf), axis=0, keepdims=True)
        return jnp.maximum(acc, jnp.where(gid_iota == gid, red, -jnp.inf))

    acc = lax.fori_loop(0, NG, step, jnp.full((NG, 128), -jnp.inf, _f32))
    out_ref[:] = jnp.maximum(out_ref[:], acc)


def _tc4(s0, s1, g, dv, b, batch_p):
    return pl.pallas_call(
        _tc4_body,
        grid=(NB,),
        in_specs=[_rspec(128), _rspec(128), _rspec(128), _rspec(8),
                  _fullspec(1, 128), _rspec(1)],
        out_specs=pl.BlockSpec((NG, 128), lambda i: (0, 0)),
        out_shape=jax.ShapeDtypeStruct((NG, 128), _f32),
    )(s0, s1, g, dv, b, batch_p)


def kernel(data, edge_index, batch, W1, b1, W2, b2, W3, b3):
    src = edge_index[0]
    dst = edge_index[1]
    data_p = jnp.pad(data, ((0, NP - N), (0, 0)))
    batch_p = jnp.pad(batch, (0, NP - N), constant_values=-1).reshape(NP, 1)
    src2 = jnp.pad(src, (0, EP2 - E), constant_values=N).reshape(2 * NT, K2, C)
    dst2 = jnp.pad(dst, (0, EP2 - E), constant_values=N).reshape(2 * NT, K2, C)
    zz128 = jnp.zeros((C, 128), _f32)
    ones_g = jnp.ones((NP, 128), _f32)

    d0, d1 = _agg_full(ones_g, dst2, dst2, zz128)
    g1, dinv = _tc1(data_p, W1, d0, d1)
    s1a, s1b = _agg_full(g1, src2, dst2, zz128)
    g2l, g2h = _tc2(s1a, s1b, g1, dinv, b1.reshape(1, 128), W2)
    s2la, s2lb = _agg_full(g2l, src2, dst2, zz128)
    s2ha, s2hb = _agg_full(g2h, src2, dst2, zz128)
    g3 = _tc3(s2la, s2lb, s2ha, s2hb, g2l, g2h, dinv, b2.reshape(1, 256), W3)
    s3a, s3b = _agg_full(g3, src2, dst2, zz128)
    return _tc4(s3a, s3b, g3, dinv, b3.reshape(1, 128), batch_p)


# speedup vs baseline: 1.0757x; 1.0757x over previous
import functools

import jax
import jax.numpy as jnp
from jax import lax
from jax.experimental import pallas as pl
from jax.experimental.pallas import tpu as pltpu
from jax.experimental.pallas import tpu_sc as plsc

N = 10000
E = 320000
NP = 10240
BR = 512
NB = NP // BR
NG = 64
C = 128
NT = 16
K2 = -(-E // (2 * NT * C)) + (-(-E // (2 * NT * C)) % 2)
EP2 = 2 * NT * K2 * C
NPS = NP // NT
W = 8

_f32 = jnp.float32


@functools.lru_cache(maxsize=None)
def _make_agg_full():
    mesh = plsc.VectorSubcoreMesh(core_axis_name="c", subcore_axis_name="s")

    def body(g, src3, dst3, zz, out0, out1, idxs_v, idxd_v, bufa,
             s_sp, gsa):
        c = lax.axis_index("c")
        s = lax.axis_index("s")
        w = c * NT + s
        pltpu.sync_copy(zz, s_sp.at[pl.ds(s * NPS, NPS)])
        pltpu.sync_copy(src3.at[w], idxs_v)
        pltpu.sync_copy(dst3.at[w], idxd_v)
        plsc.subcore_barrier()

        def step(j, carry):
            pltpu.async_copy(g.at[idxs_v.at[j]], bufa, gsa).wait()
            pltpu.sync_copy(bufa, s_sp.at[idxd_v.at[j]], add=True)
            return carry

        lax.fori_loop(0, K2, step, 0)
        plsc.subcore_barrier()

        @pl.when(c == 0)
        def _():
            pltpu.sync_copy(s_sp.at[pl.ds(s * NPS, NPS)],
                            out0.at[pl.ds(s * NPS, NPS)])

        @pl.when(c == 1)
        def _():
            pltpu.sync_copy(s_sp.at[pl.ds(s * NPS, NPS)],
                            out1.at[pl.ds(s * NPS, NPS)])

    return pl.kernel(
        body,
        out_type=[jax.ShapeDtypeStruct((NP, 128), _f32),
                  jax.ShapeDtypeStruct((NP, 128), _f32)],
        mesh=mesh,
        scratch_types=[
            pltpu.VMEM((K2, C), jnp.int32),
            pltpu.VMEM((K2, C), jnp.int32),
            pltpu.VMEM((C, 128), _f32),
            pltpu.VMEM_SHARED((NP, 128), _f32),
            pltpu.SemaphoreType.DMA,
        ],
    )


def _agg_full(g, src3, dst3, zz):
    return _make_agg_full()(g, src3, dst3, zz)


def _rspec(w):
    return pl.BlockSpec((BR, w), lambda i: (i, 0))


def _fullspec(r, w):
    return pl.BlockSpec((r, w), lambda i: (0, 0))


def _tc1_body(x_ref, w_ref, d0_ref, d1_ref, g_ref, dv_ref):
    deg = d0_ref[:, 0:1] + d1_ref[:, 0:1] + 1.0
    dinv = lax.rsqrt(jnp.maximum(deg, 1e-12))
    h = jnp.dot(x_ref[:], w_ref[:], preferred_element_type=_f32)
    g_ref[:] = dinv * h
    dv_ref[:] = jnp.broadcast_to(dinv, (BR, 8))


def _tc1(data_p, w1, d0, d1):
    return pl.pallas_call(
        _tc1_body,
        grid=(NB,),
        in_specs=[_rspec(128), _fullspec(128, 128), _rspec(128), _rspec(128)],
        out_specs=[_rspec(128), _rspec(8)],
        out_shape=[jax.ShapeDtypeStruct((NP, 128), _f32),
                   jax.ShapeDtypeStruct((NP, 8), _f32)],
    )(data_p, w1, d0, d1)


def _tc2_body(s0_ref, s1_ref, g_ref, dv_ref, b_ref, w_ref, ol_ref, oh_ref):
    dinv = dv_ref[:, 0:1]
    x = dinv * (s0_ref[:] + s1_ref[:] + g_ref[:]) + b_ref[:]
    x = jnp.maximum(x, 0.0)
    h = jnp.dot(x, w_ref[:], preferred_element_type=_f32)
    g2 = dinv * h
    ol_ref[:] = g2[:, :128]
    oh_ref[:] = g2[:, 128:]


def _tc2(s0, s1, g, dv, b, w):
    return pl.pallas_call(
        _tc2_body,
        grid=(NB,),
        in_specs=[_rspec(128), _rspec(128), _rspec(128), _rspec(8),
                  _fullspec(1, 128), _fullspec(128, 256)],
        out_specs=[_rspec(128), _rspec(128)],
        out_shape=[jax.ShapeDtypeStruct((NP, 128), _f32),
                   jax.ShapeDtypeStruct((NP, 128), _f32)],
    )(s0, s1, g, dv, b, w)


def _tc3_body(sla_ref, slb_ref, sha_ref, shb_ref, gl_ref, gh_ref, dv_ref,
              b_ref, w_ref, o_ref):
    dinv = dv_ref[:, 0:1]
    x_lo = jnp.maximum(
        dinv * (sla_ref[:] + slb_ref[:] + gl_ref[:]) + b_ref[:, :128], 0.0)
    x_hi = jnp.maximum(
        dinv * (sha_ref[:] + shb_ref[:] + gh_ref[:]) + b_ref[:, 128:], 0.0)
    h = (jnp.dot(x_lo, w_ref[:128, :], preferred_element_type=_f32)
         + jnp.dot(x_hi, w_ref[128:, :], preferred_element_type=_f32))
    o_ref[:] = dinv * h


def _tc3(sla, slb, sha, shb, gl, gh, dv, b, w):
    return pl.pallas_call(
        _tc3_body,
        grid=(NB,),
        in_specs=[_rspec(128), _rspec(128), _rspec(128), _rspec(128),
                  _rspec(128), _rspec(128), _rspec(8), _fullspec(1, 256),
                  _fullspec(256, 128)],
        out_specs=_rspec(128),
        out_shape=jax.ShapeDtypeStruct((NP, 128), _f32),
    )(sla, slb, sha, shb, gl, gh, dv, b, w)


def _tc4_body(s0_ref, s1_ref, g_ref, dv_ref, b_ref, batch_ref, out_ref):
    i = pl.program_id(0)

    @pl.when(i == 0)
    def _():
        out_ref[:] = jnp.full((NG, 128), -jnp.inf, _f32)

    dinv = dv_ref[:, 0:1]
    h = dinv * (s0_ref[:] + s1_ref[:] + g_ref[:]) + b_ref[:]
    batch = batch_ref[:]
    gid_iota = lax.broadcasted_iota(jnp.int32, (NG, 1), 0)

    def step(gid, acc):
        m = batch == gid
        red = jnp.max(jnp.where(m, h, -jnp.inf), axis=0, keepdims=True)
        return jnp.maximum(acc, jnp.where(gid_iota == gid, red, -jnp.inf))

    acc = lax.fori_loop(0, NG, step, jnp.full((NG, 128), -jnp.inf, _f32))
    out_ref[:] = jnp.maximum(out_ref[:], acc)


def _tc4(s0, s1, g, dv, b, batch_p):
    return pl.pallas_call(
        _tc4_body,
        grid=(NB,),
        in_specs=[_rspec(128), _rspec(128), _rspec(128), _rspec(8),
                  _fullspec(1, 128), _rspec(1)],
        out_specs=pl.BlockSpec((NG, 128), lambda i: (0, 0)),
        out_shape=jax.ShapeDtypeStruct((NG, 128), _f32),
    )(s0, s1, g, dv, b, batch_p)


def kernel(data, edge_index, batch, W1, b1, W2, b2, W3, b3):
    src = edge_index[0]
    dst = edge_index[1]
    data_p = jnp.pad(data, ((0, NP - N), (0, 0)))
    batch_p = jnp.pad(batch, (0, NP - N), constant_values=-1).reshape(NP, 1)
    src2 = jnp.pad(src, (0, EP2 - E), constant_values=N).reshape(2 * NT, K2, C)
    dst2 = jnp.pad(dst, (0, EP2 - E), constant_values=N).reshape(2 * NT, K2, C)
    zz128 = jnp.zeros((NPS, 128), _f32)
    ones_g = jnp.ones((NP, 128), _f32)

    d0, d1 = _agg_full(ones_g, dst2, dst2, zz128)
    g1, dinv = _tc1(data_p, W1, d0, d1)
    s1a, s1b = _agg_full(g1, src2, dst2, zz128)
    g2l, g2h = _tc2(s1a, s1b, g1, dinv, b1.reshape(1, 128), W2)
    s2la, s2lb = _agg_full(g2l, src2, dst2, zz128)
    s2ha, s2hb = _agg_full(g2h, src2, dst2, zz128)
    g3 = _tc3(s2la, s2lb, s2ha, s2hb, g2l, g2h, dinv, b2.reshape(1, 256), W3)
    s3a, s3b = _agg_full(g3, src2, dst2, zz128)
    return _tc4(s3a, s3b, g3, dinv, b3.reshape(1, 128), batch_p)


# speedup vs baseline: 1.5407x; 1.4322x over previous
import functools

import jax
import jax.numpy as jnp
from jax import lax
from jax.experimental import pallas as pl
from jax.experimental.pallas import tpu as pltpu
from jax.experimental.pallas import tpu_sc as plsc

N = 10000
E = 320000
NP = 10240
BR = 512
NB = NP // BR
NG = 64
C = 128
NT = 16
K2 = -(-E // (2 * NT * C))
EP2 = 2 * NT * K2 * C
NPS = NP // NT
W = 8

_f32 = jnp.float32


@functools.lru_cache(maxsize=None)
def _make_agg_full():
    mesh = plsc.VectorSubcoreMesh(core_axis_name="c", subcore_axis_name="s")

    def body(g, src3, dst3, zz, out0, out1, idxs_v, idxd_v, bufa,
             s_sp, gsa):
        c = lax.axis_index("c")
        s = lax.axis_index("s")
        w = c * NT + s
        pltpu.sync_copy(zz, s_sp.at[pl.ds(s * NPS, NPS)])
        pltpu.sync_copy(src3.at[w], idxs_v)
        pltpu.sync_copy(dst3.at[w], idxd_v)
        plsc.subcore_barrier()

        def step(j, carry):
            pltpu.async_copy(g.at[idxs_v.at[j]], bufa, gsa).wait()
            pltpu.sync_copy(bufa, s_sp.at[idxd_v.at[j]], add=True)
            return carry

        lax.fori_loop(0, K2, step, 0)
        plsc.subcore_barrier()

        @pl.when(c == 0)
        def _():
            pltpu.sync_copy(s_sp.at[pl.ds(s * NPS, NPS)],
                            out0.at[pl.ds(s * NPS, NPS)])

        @pl.when(c == 1)
        def _():
            pltpu.sync_copy(s_sp.at[pl.ds(s * NPS, NPS)],
                            out1.at[pl.ds(s * NPS, NPS)])

    return pl.kernel(
        body,
        out_type=[jax.ShapeDtypeStruct((NP, 128), _f32),
                  jax.ShapeDtypeStruct((NP, 128), _f32)],
        mesh=mesh,
        scratch_types=[
            pltpu.VMEM((K2, C), jnp.int32),
            pltpu.VMEM((K2, C), jnp.int32),
            pltpu.VMEM((C, 128), _f32),
            pltpu.VMEM_SHARED((NP, 128), _f32),
            pltpu.SemaphoreType.DMA,
        ],
    )


def _agg_full(g, src3, dst3, zz):
    return _make_agg_full()(g, src3, dst3, zz)


def _rspec(w):
    return pl.BlockSpec((BR, w), lambda i: (i, 0))


def _fullspec(r, w):
    return pl.BlockSpec((r, w), lambda i: (0, 0))


def _tc1_body(x_ref, w_ref, d0_ref, d1_ref, g_ref, dv_ref):
    deg = d0_ref[:, 0:1] + d1_ref[:, 0:1] + 1.0
    dinv = lax.rsqrt(jnp.maximum(deg, 1e-12))
    h = jnp.dot(x_ref[:], w_ref[:], preferred_element_type=_f32)
    g_ref[:] = dinv * h
    dv_ref[:] = jnp.broadcast_to(dinv, (BR, 8))


def _tc1(data_p, w1, d0, d1):
    return pl.pallas_call(
        _tc1_body,
        grid=(NB,),
        in_specs=[_rspec(128), _fullspec(128, 128), _rspec(128), _rspec(128)],
        out_specs=[_rspec(128), _rspec(8)],
        out_shape=[jax.ShapeDtypeStruct((NP, 128), _f32),
                   jax.ShapeDtypeStruct((NP, 8), _f32)],
    )(data_p, w1, d0, d1)


def _tc2_body(s0_ref, s1_ref, g_ref, dv_ref, b_ref, w_ref, ol_ref, oh_ref):
    dinv = dv_ref[:, 0:1]
    x = dinv * (s0_ref[:] + s1_ref[:] + g_ref[:]) + b_ref[:]
    x = jnp.maximum(x, 0.0)
    h = jnp.dot(x, w_ref[:], preferred_element_type=_f32)
    g2 = dinv * h
    ol_ref[:] = g2[:, :128]
    oh_ref[:] = g2[:, 128:]


def _tc2(s0, s1, g, dv, b, w):
    return pl.pallas_call(
        _tc2_body,
        grid=(NB,),
        in_specs=[_rspec(128), _rspec(128), _rspec(128), _rspec(8),
                  _fullspec(1, 128), _fullspec(128, 256)],
        out_specs=[_rspec(128), _rspec(128)],
        out_shape=[jax.ShapeDtypeStruct((NP, 128), _f32),
                   jax.ShapeDtypeStruct((NP, 128), _f32)],
    )(s0, s1, g, dv, b, w)


def _tc3_body(sla_ref, slb_ref, sha_ref, shb_ref, gl_ref, gh_ref, dv_ref,
              b_ref, w_ref, o_ref):
    dinv = dv_ref[:, 0:1]
    x_lo = jnp.maximum(
        dinv * (sla_ref[:] + slb_ref[:] + gl_ref[:]) + b_ref[:, :128], 0.0)
    x_hi = jnp.maximum(
        dinv * (sha_ref[:] + shb_ref[:] + gh_ref[:]) + b_ref[:, 128:], 0.0)
    h = (jnp.dot(x_lo, w_ref[:128, :], preferred_element_type=_f32)
         + jnp.dot(x_hi, w_ref[128:, :], preferred_element_type=_f32))
    o_ref[:] = dinv * h


def _tc3(sla, slb, sha, shb, gl, gh, dv, b, w):
    return pl.pallas_call(
        _tc3_body,
        grid=(NB,),
        in_specs=[_rspec(128), _rspec(128), _rspec(128), _rspec(128),
                  _rspec(128), _rspec(128), _rspec(8), _fullspec(1, 256),
                  _fullspec(256, 128)],
        out_specs=_rspec(128),
        out_shape=jax.ShapeDtypeStruct((NP, 128), _f32),
    )(sla, slb, sha, shb, gl, gh, dv, b, w)


def _tc4_body(s0_ref, s1_ref, g_ref, dv_ref, b_ref, batch_ref, out_ref):
    i = pl.program_id(0)

    @pl.when(i == 0)
    def _():
        out_ref[:] = jnp.full((NG, 128), -jnp.inf, _f32)

    dinv = dv_ref[:, 0:1]
    h = dinv * (s0_ref[:] + s1_ref[:] + g_ref[:]) + b_ref[:]
    batch = batch_ref[:]
    gid_iota = lax.broadcasted_iota(jnp.int32, (NG, 1), 0)

    def step(gid, acc):
        m = batch == gid
        red = jnp.max(jnp.where(m, h, -jnp.inf), axis=0, keepdims=True)
        return jnp.maximum(acc, jnp.where(gid_iota == gid, red, -jnp.inf))

    acc = lax.fori_loop(0, NG, step, jnp.full((NG, 128), -jnp.inf, _f32))
    out_ref[:] = jnp.maximum(out_ref[:], acc)


def _tc4(s0, s1, g, dv, b, batch_p):
    return pl.pallas_call(
        _tc4_body,
        grid=(NB,),
        in_specs=[_rspec(128), _rspec(128), _rspec(128), _rspec(8),
                  _fullspec(1, 128), _rspec(1)],
        out_specs=pl.BlockSpec((NG, 128), lambda i: (0, 0)),
        out_shape=jax.ShapeDtypeStruct((NG, 128), _f32),
    )(s0, s1, g, dv, b, batch_p)


def kernel(data, edge_index, batch, W1, b1, W2, b2, W3, b3):
    src = edge_index[0]
    dst = edge_index[1]
    data_p = jnp.pad(data, ((0, NP - N), (0, 0)))
    batch_p = jnp.pad(batch, (0, NP - N), constant_values=-1).reshape(NP, 1)
    src2 = jnp.pad(src, (0, EP2 - E), constant_values=N).reshape(2 * NT, K2, C)
    dst2 = jnp.pad(dst, (0, EP2 - E), constant_values=N).reshape(2 * NT, K2, C)
    zz128 = jnp.zeros((NPS, 128), _f32)
    ones_g = jnp.ones((NP, 128), _f32)

    d0, d1 = _agg_full(ones_g, src2, dst2, zz128)
    g1, dinv = _tc1(data_p, W1, d0, d1)
    s1a, s1b = _agg_full(g1, src2, dst2, zz128)
    g2l, g2h = _tc2(s1a, s1b, g1, dinv, b1.reshape(1, 128), W2)
    s2la, s2lb = _agg_full(g2l, src2, dst2, zz128)
    s2ha, s2hb = _agg_full(g2h, src2, dst2, zz128)
    g3 = _tc3(s2la, s2lb, s2ha, s2hb, g2l, g2h, dinv, b2.reshape(1, 256), W3)
    s3a, s3b = _agg_full(g3, src2, dst2, zz128)
    return _tc4(s3a, s3b, g3, dinv, b3.reshape(1, 128), batch_p)
